# baseline (speedup 1.0000x reference)
import functools

import jax
import jax.numpy as jnp
from jax.experimental import pallas as pl
from jax.experimental.pallas import tpu as pltpu

_TAU1 = 1.0
_NUM_K = 3
_NUM_LABELS = 64
_U = 192.0
_SHIFT = 36864.0
_BM = 512


def _loss_body(e_ref, labr_ref, labcf_ref, out_ref,
               aaug_ref, eaug_ref, cnt_ref, *, batch, dim):
    lab_row = labr_ref[:, :]
    lab_cf = labcf_ref[:, :]
    bins_col = jax.lax.broadcasted_iota(
        jnp.int32, (_NUM_LABELS, 1), 0
    ).astype(jnp.float32)
    bins_row = jax.lax.broadcasted_iota(
        jnp.int32, (1, _NUM_LABELS), 1
    ).astype(jnp.float32)
    onehot_all = jnp.where(bins_col == lab_row, 1.0, 0.0)
    counts = jnp.sum(onehot_all, axis=1, keepdims=True)
    oh_full = jnp.where(lab_cf == bins_row, 1.0, 0.0)
    cnt_ref[:, :] = jax.lax.dot_general(
        oh_full, counts, (((1,), (0,)), ((), ())),
        preferred_element_type=jnp.float32,
    )
    e = e_ref[:, :]
    aaug_ref[:, :dim] = (e * 0.5).astype(jnp.bfloat16)
    aaug_ref[:, dim:] = (oh_full * (-_U)).astype(jnp.bfloat16)
    eaug_ref[:, :dim] = e.astype(jnp.bfloat16)
    eaug_ref[:, dim:] = (oh_full * _U).astype(jnp.bfloat16)

    total = jnp.zeros((1, 1), jnp.float32)
    for c in range(batch // _BM):
        rows = pl.ds(c * _BM, _BM)
        sim_s = jax.lax.dot_general(
            aaug_ref[rows, :], eaug_ref[:, :], (((1,), (1,)), ((), ())),
            preferred_element_type=jnp.float32,
        )

        max_s = jnp.max(sim_s, axis=1, keepdims=True)
        t = jnp.tanh((max_s - _SHIFT) - _TAU1 * sim_s)
        sum_all = jnp.sum(t, axis=1, keepdims=True)

        cnt = cnt_ref[rows, :]
        sum_t = sum_all + (jnp.float32(batch) - cnt)
        total += jnp.sum(
            0.5 - 0.5 * sum_t / cnt, axis=0, keepdims=True
        ) * (float(_NUM_K) / batch)

    out_ref[:, :] = total


def kernel(embeddings, labels):
    batch, dim = embeddings.shape
    lab_f = labels.astype(jnp.float32)
    labels_row = lab_f.reshape(1, batch)
    labels_col = lab_f.reshape(batch, 1)
    out = pl.pallas_call(
        functools.partial(_loss_body, batch=batch, dim=dim),
        grid=(1,),
        in_specs=[
            pl.BlockSpec((batch, dim), lambda i: (0, 0)),
            pl.BlockSpec((1, batch), lambda i: (0, 0)),
            pl.BlockSpec((batch, 1), lambda i: (0, 0)),
        ],
        out_specs=pl.BlockSpec((1, 1), lambda i: (0, 0)),
        out_shape=jax.ShapeDtypeStruct((1, 1), jnp.float32),
        scratch_shapes=[
            pltpu.VMEM((batch, dim + _NUM_LABELS), jnp.bfloat16),
            pltpu.VMEM((batch, dim + _NUM_LABELS), jnp.bfloat16),
            pltpu.VMEM((batch, 1), jnp.float32),
        ],
    )(embeddings, labels_row, labels_col)
    return out[0, 0]
